# Initial kernel scaffold; baseline (speedup 1.0000x reference)
import jax
import jax.numpy as jnp
from jax.experimental import pallas as pl


def kernel(x, target):
    raise NotImplementedError("write your pallas kernel here")


# speedup vs baseline: 2.2863x; 2.2863x over previous
import functools
import math

import jax
import jax.numpy as jnp
from jax.experimental import pallas as pl
from jax.experimental.pallas import tpu as pltpu

_PAD_IDX = 0
_SMOOTHING = 0.1
_BLK = 2048


def _loss_body(nvb, vocab, fill, conf, c_row, x_ref, tgt_ref, out_ref):
    j = pl.program_id(0)
    blk = x_ref.shape[1]
    cols = j * blk + jax.lax.broadcasted_iota(jnp.int32, (1, blk), 1)
    tgt = tgt_ref[:, :]
    nonpad = (tgt != _PAD_IDX).astype(jnp.float32)
    w = jnp.where(cols == _PAD_IDX, 0.0, -fill)
    w = jnp.where(cols == tgt, -conf, w)
    wx = (w * nonpad) * x_ref[:, :]
    contrib = jnp.sum(jnp.where(cols < vocab, wx, 0.0))

    @pl.when(j == 0)
    def _init():
        out_ref[0, 0] = c_row * jnp.sum(nonpad) + contrib

    @pl.when(j != 0)
    def _acc():
        out_ref[0, 0] += contrib


def kernel(x, target):
    n, vocab = x.shape
    fill = _SMOOTHING / (vocab - 2)
    conf = 1.0 - _SMOOTHING
    c_row = (vocab - 2) * fill * math.log(fill) + conf * math.log(conf)
    nvb = pl.cdiv(vocab, _BLK)
    tgt2d = target.reshape(n, 1)
    out = pl.pallas_call(
        functools.partial(_loss_body, nvb, vocab, fill, conf, c_row),
        grid=(nvb,),
        in_specs=[
            pl.BlockSpec((n, _BLK), lambda j: (0, j)),
            pl.BlockSpec((n, 1), lambda j: (0, 0)),
        ],
        out_specs=pl.BlockSpec(memory_space=pltpu.SMEM),
        out_shape=jax.ShapeDtypeStruct((1, 1), jnp.float32),
    )(x, tgt2d)
    return out.reshape(1)


# speedup vs baseline: 2.3250x; 1.0169x over previous
import functools
import math

import jax
import jax.numpy as jnp
from jax import lax
from jax.experimental import pallas as pl
from jax.experimental.pallas import tpu as pltpu
from jax.experimental.pallas import tpu_sc as plsc

_PAD_IDX = 0
_SMOOTHING = 0.1
_BLK = 2048
_LANES = 128


def _sc_gather_body(vocab, n_rows, rows_per_worker, num_cores,
                    x_hbm, tgt_hbm, out_hbm, tgt_v, buf_v, out_v, sem):
    wid = lax.axis_index("s") * num_cores + lax.axis_index("c")
    base = wid * rows_per_worker
    pltpu.sync_copy(tgt_hbm, tgt_v)
    lane = lax.iota(jnp.int32, 16)
    copies = []
    for g in range(rows_per_worker // 16):
        tv = tgt_v[pl.ds(base + g * 16, 16)]
        for j in range(16):
            k = g * 16 + j
            t_k = tv[j]
            colg = pl.multiple_of(t_k & ~127, 128)
            rowg = pl.multiple_of(base + (k // 8) * 8, 8)
            cp = pltpu.make_async_copy(
                x_hbm.at[pl.ds(rowg, 8), pl.ds(colg, 128)],
                buf_v.at[pl.ds(k * 8, 8)], sem)
            cp.start()
            copies.append(cp)
    for cp in copies:
        cp.wait()
    for g in range(rows_per_worker // 16):
        tvec = tgt_v[pl.ds(base + g * 16, 16)]
        vals = plsc.load_gather(
            buf_v, [(lane + g * 16) * 8 + (lane & 7), tvec & 127])
        vals = jnp.where(tvec != _PAD_IDX, vals, 0.0)
        out_v[pl.ds(g * 16, 16)] = vals
    pltpu.sync_copy(out_v, out_hbm.at[pl.ds(base, rows_per_worker)])


def _sc_gather(x, target):
    n, vocab = x.shape
    info = plsc.get_sparse_core_info()
    nw = info.num_cores * info.num_subcores
    rpw = n // nw
    mesh = plsc.VectorSubcoreMesh(core_axis_name="c", subcore_axis_name="s")
    k = functools.partial(
        pl.kernel,
        mesh=mesh,
        out_type=jax.ShapeDtypeStruct((n,), jnp.float32),
        scratch_types=[
            pltpu.VMEM((n,), jnp.int32),
            pltpu.VMEM((rpw * 8, 128), jnp.float32),
            pltpu.VMEM((rpw,), jnp.float32),
            pltpu.SemaphoreType.DMA,
        ],
        compiler_params=pltpu.CompilerParams(needs_layout_passes=False),
    )(functools.partial(_sc_gather_body, vocab, n, rpw, info.num_cores))
    return k(x, target)


def _loss_body(nvb, vocab, fill, conf, c_row,
               x_ref, tgt_ref, gat_ref, out_ref, acc_ref):
    j = pl.program_id(0)
    blk = x_ref.shape[1]
    nchunk = blk // _LANES

    def chunk_sum(xb):
        s = xb[:, 0:_LANES]
        for k in range(1, nchunk):
            s = s + xb[:, k * _LANES:(k + 1) * _LANES]
        return s

    @pl.when(j == 0)
    def _first():
        xb = x_ref[:, :]
        acc_ref[:, :] = chunk_sum(xb)
        tgt = tgt_ref[:, :]
        nonpad = (tgt != _PAD_IDX).astype(jnp.float32)
        cnt = jnp.sum(nonpad)
        s0 = jnp.sum(nonpad * xb[:, 0:1])
        t_sum = jnp.sum(gat_ref[:, :])
        out_ref[0, 0] = c_row * cnt + fill * s0 + (fill - conf) * t_sum

    @pl.when(jnp.logical_and(j > 0, j < nvb - 1))
    def _mid():
        acc_ref[:, :] += chunk_sum(x_ref[:, :])

    @pl.when(j == nvb - 1)
    def _last():
        cols = jax.lax.broadcasted_iota(jnp.int32, (1, blk), 1)
        limit = vocab - j * blk
        xb = jnp.where(cols < limit, x_ref[:, :], 0.0)
        acc = acc_ref[:, :] + chunk_sum(xb)
        nonpad = (tgt_ref[:, :] != _PAD_IDX).astype(jnp.float32)
        out_ref[0, 0] += -fill * jnp.sum(nonpad * acc)


def kernel(x, target):
    n, vocab = x.shape
    fill = _SMOOTHING / (vocab - 2)
    conf = 1.0 - _SMOOTHING
    c_row = (vocab - 2) * fill * math.log(fill) + conf * math.log(conf)
    nvb = pl.cdiv(vocab, _BLK)
    gat = _sc_gather(x, target)
    tgt2d = target.reshape(n, 1)
    gat2d = gat.reshape(n, 1)
    out = pl.pallas_call(
        functools.partial(_loss_body, nvb, vocab, fill, conf, c_row),
        grid=(nvb,),
        in_specs=[
            pl.BlockSpec((n, _BLK), lambda j: (0, j)),
            pl.BlockSpec((n, 1), lambda j: (0, 0)),
            pl.BlockSpec((n, 1), lambda j: (0, 0)),
        ],
        out_specs=pl.BlockSpec(memory_space=pltpu.SMEM),
        out_shape=jax.ShapeDtypeStruct((1, 1), jnp.float32),
        scratch_shapes=[pltpu.VMEM((n, _LANES), jnp.float32)],
    )(x, tgt2d, gat2d)
    return out.reshape(1)
